# initial kernel scaffold (unmeasured)
import jax
import jax.numpy as jnp
from jax import lax
from jax.experimental import pallas as pl
from jax.experimental.pallas import tpu as pltpu


def kernel(
    t,
):
    def body(*refs):
        pass

    out_shape = jax.ShapeDtypeStruct(..., jnp.float32)
    return pl.pallas_call(body, out_shape=out_shape)(...)


# baseline (device time: 158897 ns/iter reference)
import jax
import jax.numpy as jnp
from jax import lax
from jax.experimental import pallas as pl
from jax.experimental.pallas import tpu as pltpu

N_DEV = 4


def kernel(t):
    m, n = t.shape
    ch = m // N_DEV

    def body(t_ref, out_ref, sb, rb, rs_send_sems, rs_recv_sems,
             ag_send_sems, ag_recv_sems):
        my = lax.axis_index("i")
        left = lax.rem(my + N_DEV - 1, N_DEV)
        right = lax.rem(my + 1, N_DEV)

        barrier_sem = pltpu.get_barrier_semaphore()
        for nbr in (left, right):
            pl.semaphore_signal(
                barrier_sem, inc=1,
                device_id=(nbr,), device_id_type=pl.DeviceIdType.MESH,
            )
        pl.semaphore_wait(barrier_sem, 2)

        def t_chunk(idx):
            return t_ref[pl.ds(idx * ch, ch), :]

        sb[0, :, :] = t_chunk(my).astype(jnp.bfloat16)
        for h in range(N_DEV - 1):
            rdma = pltpu.make_async_remote_copy(
                src_ref=sb.at[h],
                dst_ref=rb.at[h],
                send_sem=rs_send_sems.at[h],
                recv_sem=rs_recv_sems.at[h],
                device_id=(right,),
                device_id_type=pl.DeviceIdType.MESH,
            )
            rdma.start()
            rdma.wait()
            idx = lax.rem(my + N_DEV - 1 - h, N_DEV)
            acc = rb[h, :, :].astype(jnp.float32) + t_chunk(idx)
            if h < N_DEV - 2:
                sb[h + 1, :, :] = acc.astype(jnp.bfloat16)
            else:
                s = acc
                r = jnp.maximum(s, 0.0)
                fval = (jnp.tanh(s) * s * s + r * r * r).astype(jnp.bfloat16)
                own = lax.rem(my + 1, N_DEV)
                out_ref[pl.ds(own * ch, ch), :] = fval

        for h in range(N_DEV - 1):
            idx_send = lax.rem(my + 1 - h + N_DEV, N_DEV)
            rdma = pltpu.make_async_remote_copy(
                src_ref=out_ref.at[pl.ds(idx_send * ch, ch), :],
                dst_ref=out_ref.at[pl.ds(idx_send * ch, ch), :],
                send_sem=ag_send_sems.at[h],
                recv_sem=ag_recv_sems.at[h],
                device_id=(right,),
                device_id_type=pl.DeviceIdType.MESH,
            )
            rdma.start()
            rdma.wait()

    return pl.pallas_call(
        body,
        out_shape=jax.ShapeDtypeStruct((m, n), jnp.bfloat16),
        in_specs=[pl.BlockSpec(memory_space=pltpu.VMEM)],
        out_specs=pl.BlockSpec(memory_space=pltpu.VMEM),
        scratch_shapes=[
            pltpu.VMEM((N_DEV - 1, ch, n), jnp.bfloat16),
            pltpu.VMEM((N_DEV - 1, ch, n), jnp.bfloat16),
            pltpu.SemaphoreType.DMA((N_DEV - 1,)),
            pltpu.SemaphoreType.DMA((N_DEV - 1,)),
            pltpu.SemaphoreType.DMA((N_DEV - 1,)),
            pltpu.SemaphoreType.DMA((N_DEV - 1,)),
        ],
        compiler_params=pltpu.CompilerParams(collective_id=0),
    )(t)


# device time: 91626 ns/iter; 1.7342x vs baseline; 1.7342x over previous
import jax
import jax.numpy as jnp
from jax import lax
from jax.experimental import pallas as pl
from jax.experimental.pallas import tpu as pltpu

N_DEV = 4


def kernel(t):
    m, n = t.shape
    ch = m // N_DEV
    hf = ch // 2

    def body(t_ref, out_ref, sb_cw, sb_ccw, rb_cw, rb_ccw,
             rs_cw_s, rs_cw_r, rs_ccw_s, rs_ccw_r,
             ag_cw_s, ag_cw_r, ag_ccw_s, ag_ccw_r):
        my = lax.axis_index("i")
        left = lax.rem(my + N_DEV - 1, N_DEV)
        right = lax.rem(my + 1, N_DEV)

        barrier_sem = pltpu.get_barrier_semaphore()
        for nbr in (left, right):
            pl.semaphore_signal(
                barrier_sem, inc=1,
                device_id=(nbr,), device_id_type=pl.DeviceIdType.MESH,
            )
        pl.semaphore_wait(barrier_sem, 2)

        def t_top(idx):
            return t_ref[pl.ds(idx * ch, hf), :]

        def t_bot(idx):
            return t_ref[pl.ds(idx * ch + hf, hf), :]

        sb_cw[0, :, :] = t_top(my).astype(jnp.bfloat16)
        sb_ccw[0, :, :] = t_bot(lax.rem(my + 2, N_DEV)).astype(jnp.bfloat16)
        for h in range(N_DEV - 1):
            r_cw = pltpu.make_async_remote_copy(
                src_ref=sb_cw.at[h], dst_ref=rb_cw.at[h],
                send_sem=rs_cw_s.at[h], recv_sem=rs_cw_r.at[h],
                device_id=(right,), device_id_type=pl.DeviceIdType.MESH,
            )
            r_ccw = pltpu.make_async_remote_copy(
                src_ref=sb_ccw.at[h], dst_ref=rb_ccw.at[h],
                send_sem=rs_ccw_s.at[h], recv_sem=rs_ccw_r.at[h],
                device_id=(left,), device_id_type=pl.DeviceIdType.MESH,
            )
            r_cw.start()
            r_ccw.start()
            r_cw.wait()
            r_ccw.wait()
            idx_cw = lax.rem(my + N_DEV - 1 - h, N_DEV)
            idx_ccw = lax.rem(my + 3 + h, N_DEV)
            acc_cw = rb_cw[h, :, :].astype(jnp.float32) + t_top(idx_cw)
            acc_ccw = rb_ccw[h, :, :].astype(jnp.float32) + t_bot(idx_ccw)
            if h < N_DEV - 2:
                sb_cw[h + 1, :, :] = acc_cw.astype(jnp.bfloat16)
                sb_ccw[h + 1, :, :] = acc_ccw.astype(jnp.bfloat16)
            else:
                own = lax.rem(my + 1, N_DEV)
                for s, row0 in ((acc_cw, own * ch), (acc_ccw, own * ch + hf)):
                    r = jnp.maximum(s, 0.0)
                    fval = (jnp.tanh(s) * s * s + r * r * r)
                    out_ref[pl.ds(row0, hf), :] = fval.astype(jnp.bfloat16)

        for h in range(N_DEV - 1):
            i_cw = lax.rem(my + 1 - h + N_DEV, N_DEV)
            i_ccw = lax.rem(my + 1 + h, N_DEV)
            a_cw = pltpu.make_async_remote_copy(
                src_ref=out_ref.at[pl.ds(i_cw * ch, hf), :],
                dst_ref=out_ref.at[pl.ds(i_cw * ch, hf), :],
                send_sem=ag_cw_s.at[h], recv_sem=ag_cw_r.at[h],
                device_id=(right,), device_id_type=pl.DeviceIdType.MESH,
            )
            a_ccw = pltpu.make_async_remote_copy(
                src_ref=out_ref.at[pl.ds(i_ccw * ch + hf, hf), :],
                dst_ref=out_ref.at[pl.ds(i_ccw * ch + hf, hf), :],
                send_sem=ag_ccw_s.at[h], recv_sem=ag_ccw_r.at[h],
                device_id=(left,), device_id_type=pl.DeviceIdType.MESH,
            )
            a_cw.start()
            a_ccw.start()
            a_cw.wait()
            a_ccw.wait()

    nh = N_DEV - 1
    return pl.pallas_call(
        body,
        out_shape=jax.ShapeDtypeStruct((m, n), jnp.bfloat16),
        in_specs=[pl.BlockSpec(memory_space=pltpu.VMEM)],
        out_specs=pl.BlockSpec(memory_space=pltpu.VMEM),
        scratch_shapes=[
            pltpu.VMEM((nh, hf, n), jnp.bfloat16),
            pltpu.VMEM((nh, hf, n), jnp.bfloat16),
            pltpu.VMEM((nh, hf, n), jnp.bfloat16),
            pltpu.VMEM((nh, hf, n), jnp.bfloat16),
            pltpu.SemaphoreType.DMA((nh,)),
            pltpu.SemaphoreType.DMA((nh,)),
            pltpu.SemaphoreType.DMA((nh,)),
            pltpu.SemaphoreType.DMA((nh,)),
            pltpu.SemaphoreType.DMA((nh,)),
            pltpu.SemaphoreType.DMA((nh,)),
            pltpu.SemaphoreType.DMA((nh,)),
            pltpu.SemaphoreType.DMA((nh,)),
        ],
        compiler_params=pltpu.CompilerParams(collective_id=0),
    )(t)


# device time: 81717 ns/iter; 1.9445x vs baseline; 1.1213x over previous
import jax
import jax.numpy as jnp
from jax import lax
from jax.experimental import pallas as pl
from jax.experimental.pallas import tpu as pltpu

N_DEV = 4
SEG = 2


def kernel(t):
    m, n = t.shape
    ch = m // N_DEV
    hf = ch // 2
    sg = hf // SEG
    nh = N_DEV - 1

    def body(t_ref, out_ref, sb_cw, sb_ccw, rb_cw, rb_ccw,
             rs_cw_s, rs_cw_r, rs_ccw_s, rs_ccw_r,
             ag_cw_s, ag_cw_r, ag_ccw_s, ag_ccw_r):
        my = lax.axis_index("i")
        left = lax.rem(my + N_DEV - 1, N_DEV)
        right = lax.rem(my + 1, N_DEV)

        barrier_sem = pltpu.get_barrier_semaphore()
        for nbr in (left, right):
            pl.semaphore_signal(
                barrier_sem, inc=1,
                device_id=(nbr,), device_id_type=pl.DeviceIdType.MESH,
            )
        pl.semaphore_wait(barrier_sem, 2)

        def t_seg(idx, d, g):
            return t_ref[pl.ds(idx * ch + d * hf + g * sg, sg), :]

        def rs_rdma(d, h, g):
            sb, rb = (sb_cw, rb_cw) if d == 0 else (sb_ccw, rb_ccw)
            ss, rs_ = (rs_cw_s, rs_cw_r) if d == 0 else (rs_ccw_s, rs_ccw_r)
            tgt = right if d == 0 else left
            return pltpu.make_async_remote_copy(
                src_ref=sb.at[h, pl.ds(g * sg, sg), :],
                dst_ref=rb.at[h, pl.ds(g * sg, sg), :],
                send_sem=ss.at[h, g], recv_sem=rs_.at[h, g],
                device_id=(tgt,), device_id_type=pl.DeviceIdType.MESH,
            )

        def ag_rows(d, h, g):
            off = -h if d == 0 else h
            idx = lax.rem(my + 1 + off + N_DEV, N_DEV)
            return idx * ch + d * hf + g * sg

        def ag_rdma(d, h, g):
            ss, rs_ = (ag_cw_s, ag_cw_r) if d == 0 else (ag_ccw_s, ag_ccw_r)
            tgt = right if d == 0 else left
            row0 = ag_rows(d, h, g)
            return pltpu.make_async_remote_copy(
                src_ref=out_ref.at[pl.ds(row0, sg), :],
                dst_ref=out_ref.at[pl.ds(row0, sg), :],
                send_sem=ss.at[h, g], recv_sem=rs_.at[h, g],
                device_id=(tgt,), device_id_type=pl.DeviceIdType.MESH,
            )

        start_cw = my
        start_ccw = lax.rem(my + 2, N_DEV)
        for g in range(SEG):
            for d, idx0 in ((0, start_cw), (1, start_ccw)):
                sb = sb_cw if d == 0 else sb_ccw
                sb[0, pl.ds(g * sg, sg), :] = (
                    t_seg(idx0, d, g).astype(jnp.bfloat16))
                rs_rdma(d, h=0, g=g).start()

        own = lax.rem(my + 1, N_DEV)
        for h in range(nh):
            for g in range(SEG):
                for d in (0, 1):
                    rb = rb_cw if d == 0 else rb_ccw
                    idx = (lax.rem(my + N_DEV - 1 - h, N_DEV) if d == 0
                           else lax.rem(my + 3 + h, N_DEV))
                    rs_rdma(d, h, g).wait_recv()
                    acc = (rb[h, pl.ds(g * sg, sg), :].astype(jnp.float32)
                           + t_seg(idx, d, g))
                    if h < nh - 1:
                        sb = sb_cw if d == 0 else sb_ccw
                        sb[h + 1, pl.ds(g * sg, sg), :] = (
                            acc.astype(jnp.bfloat16))
                        rs_rdma(d, h + 1, g).start()
                    else:
                        r = jnp.maximum(acc, 0.0)
                        fval = jnp.tanh(acc) * acc * acc + r * r * r
                        row0 = own * ch + d * hf + g * sg
                        out_ref[pl.ds(row0, sg), :] = fval.astype(jnp.bfloat16)
                        ag_rdma(d, h=0, g=g).start()

        for h in range(nh):
            for g in range(SEG):
                for d in (0, 1):
                    ag_rdma(d, h, g).wait_recv()
                    if h < nh - 1:
                        ag_rdma(d, h + 1, g).start()

        for h in range(nh):
            for g in range(SEG):
                for d in (0, 1):
                    rs_rdma(d, h, g).wait_send()
                    ag_rdma(d, h, g).wait_send()

    return pl.pallas_call(
        body,
        out_shape=jax.ShapeDtypeStruct((m, n), jnp.bfloat16),
        in_specs=[pl.BlockSpec(memory_space=pltpu.VMEM)],
        out_specs=pl.BlockSpec(memory_space=pltpu.VMEM),
        scratch_shapes=[
            pltpu.VMEM((nh, hf, n), jnp.bfloat16),
            pltpu.VMEM((nh, hf, n), jnp.bfloat16),
            pltpu.VMEM((nh, hf, n), jnp.bfloat16),
            pltpu.VMEM((nh, hf, n), jnp.bfloat16),
            pltpu.SemaphoreType.DMA((nh, SEG)),
            pltpu.SemaphoreType.DMA((nh, SEG)),
            pltpu.SemaphoreType.DMA((nh, SEG)),
            pltpu.SemaphoreType.DMA((nh, SEG)),
            pltpu.SemaphoreType.DMA((nh, SEG)),
            pltpu.SemaphoreType.DMA((nh, SEG)),
            pltpu.SemaphoreType.DMA((nh, SEG)),
            pltpu.SemaphoreType.DMA((nh, SEG)),
        ],
        compiler_params=pltpu.CompilerParams(collective_id=0),
    )(t)
